# baseline (speedup 1.0000x reference)
import functools

import jax
import jax.numpy as jnp
from jax import lax
from jax.experimental import pallas as pl
from jax.experimental.pallas import tpu as pltpu
from jax.experimental.pallas import tpu_sc as plsc

B = 4096
T = 64
EMBED_DIM = 1024
GB = 128
NC, NS = 2, 16
NW = NC * NS
GPW = B // NW
L = 16
NV = T // L


def _sp_body(z_ref, xc_ref, sp_ref):
    z = z_ref[...]
    xc = xc_ref[...]
    key = jnp.where(xc == 0, z, jnp.inf)
    ki = key[:, None, :]
    kj = key[None, :, :]
    ii = lax.broadcasted_iota(jnp.int32, (T, T, GB), 0)
    jj = lax.broadcasted_iota(jnp.int32, (T, T, GB), 1)
    before = (kj < ki) | ((kj == ki) & (jj < ii))
    rank = jnp.sum(before.astype(jnp.int32), axis=1)
    sp_t = jnp.sum(jnp.where(rank[None, :, :] == ii, jj, 0), axis=1)
    sp_ref[...] = sp_t.T


def _tc_sp(z2d_t, xc_t):
    return pl.pallas_call(
        _sp_body,
        grid=(B // GB,),
        in_specs=[
            pl.BlockSpec((T, GB), lambda i: (0, i)),
            pl.BlockSpec((T, GB), lambda i: (0, i)),
        ],
        out_specs=pl.BlockSpec((GB, T), lambda i: (i, 0)),
        out_shape=jax.ShapeDtypeStruct((B, T), jnp.int32),
    )(z2d_t, xc_t)


_GDN = lax.GatherDimensionNumbers(
    offset_dims=(), collapsed_slice_dims=(0,), start_index_map=(0,)
)


def _permute(x, idx):
    return lax.gather(
        x, idx[:, None], _GDN, (1,),
        mode=lax.GatherScatterMode.PROMISE_IN_BOUNDS,
    )


@functools.lru_cache(maxsize=1)
def _get_sc_kernel():
    mesh = plsc.VectorSubcoreMesh(
        core_axis_name="c", subcore_axis_name="s", num_cores=NC, num_subcores=NS
    )

    @functools.partial(
        pl.kernel,
        out_type=jax.ShapeDtypeStruct((B * T, EMBED_DIM), jnp.float32),
        mesh=mesh,
        scratch_types=[
            pltpu.VMEM((T, EMBED_DIM), jnp.float32),
            pltpu.VMEM((GPW, T), jnp.float32),
            pltpu.VMEM((GPW, T), jnp.int32),
            pltpu.VMEM((GPW, T), jnp.int32),
            pltpu.SemaphoreType.DMA,
        ],
    )
    def _sc_emit(z_hbm, xc_hbm, table_hbm, emb_hbm,
                 table_v, z_v, xc_v, dest_v, sem):
        wid = lax.axis_index("s") * NC + lax.axis_index("c")
        base = wid * GPW
        pltpu.sync_copy(table_hbm, table_v)
        pltpu.sync_copy(z_hbm.at[pl.ds(base, GPW)], z_v)
        pltpu.sync_copy(xc_hbm.at[pl.ds(base, GPW)], xc_v)

        iota = lax.iota(jnp.int32, L)
        rot_idx = [(iota + r) & (L - 1) for r in range(L)]
        rot_lt = [((iota + r) & (L - 1)) < iota for r in range(L)]

        def graph_body(g, carry):
            ks = []
            for c in range(NV):
                zc = z_v[g, pl.ds(c * L, L)]
                xcc = xc_v[g, pl.ds(c * L, L)]
                ks.append(jnp.where(xcc == 0, zc, jnp.inf))
            ranks = [jnp.zeros((L,), jnp.int32) for _ in range(NV)]
            for r in range(L):
                kdr = [_permute(ks[d], rot_idx[r]) for d in range(NV)]
                for c in range(NV):
                    for d in range(NV):
                        if d < c:
                            cond = kdr[d] <= ks[c]
                        elif d > c:
                            cond = kdr[d] < ks[c]
                        else:
                            if r == 0:
                                continue
                            cond = (kdr[d] < ks[c]) | (
                                (kdr[d] == ks[c]) & rot_lt[r]
                            )
                        ranks[c] = ranks[c] + jnp.where(cond, 1, 0)
            rowbase = (base + g) * T
            for c in range(NV):
                dest_v[g, pl.ds(c * L, L)] = ranks[c] + rowbase
            pltpu.async_copy(table_v, emb_hbm.at[dest_v.at[g]], sem)
            return carry

        lax.fori_loop(0, GPW, graph_body, 0)

        def drain_body(g, carry):
            pltpu.make_async_copy(table_v, emb_hbm.at[dest_v.at[g]], sem).wait()
            return carry

        lax.fori_loop(0, GPW, drain_body, 0)

    return _sc_emit


def kernel(pos_clone, x, table):
    z2d = pos_clone[:, :, 2]
    xc = x[:, :, 0].astype(jnp.int32)
    emb_flat = _get_sc_kernel()(z2d, xc, table)
    sp = _tc_sp(z2d.T, xc.T)
    return (emb_flat.reshape(B, T, EMBED_DIM), sp)


# speedup vs baseline: 1.0019x; 1.0019x over previous
import functools

import jax
import jax.numpy as jnp
from jax import lax
from jax.experimental import pallas as pl
from jax.experimental.pallas import tpu as pltpu
from jax.experimental.pallas import tpu_sc as plsc

B = 4096
T = 64
EMBED_DIM = 1024
GB = 128
NC, NS = 2, 16
NW = NC * NS
GPW = B // NW
L = 16
NV = T // L


def _sp_body(z_ref, xc_ref, sp_ref):
    z = z_ref[...]
    xc = xc_ref[...]
    key = jnp.where(xc == 0, z, jnp.inf)
    ki = key[:, None, :]
    kj = key[None, :, :]
    ii = lax.broadcasted_iota(jnp.int32, (T, T, GB), 0)
    jj = lax.broadcasted_iota(jnp.int32, (T, T, GB), 1)
    before = (kj < ki) | ((kj == ki) & (jj < ii))
    rank = jnp.sum(before.astype(jnp.int32), axis=1)
    sp_t = jnp.sum(jnp.where(rank[None, :, :] == ii, jj, 0), axis=1)
    sp_ref[...] = sp_t.T


def _tc_sp(z2d_t, xc_t):
    return pl.pallas_call(
        _sp_body,
        grid=(B // GB,),
        in_specs=[
            pl.BlockSpec((T, GB), lambda i: (0, i)),
            pl.BlockSpec((T, GB), lambda i: (0, i)),
        ],
        out_specs=pl.BlockSpec((GB, T), lambda i: (i, 0)),
        out_shape=jax.ShapeDtypeStruct((B, T), jnp.int32),
    )(z2d_t, xc_t)


_GDN = lax.GatherDimensionNumbers(
    offset_dims=(), collapsed_slice_dims=(0,), start_index_map=(0,)
)


def _permute(x, idx):
    return lax.gather(
        x, idx[:, None], _GDN, (1,),
        mode=lax.GatherScatterMode.PROMISE_IN_BOUNDS,
    )


@functools.lru_cache(maxsize=1)
def _get_sc_kernel():
    mesh = plsc.VectorSubcoreMesh(
        core_axis_name="c", subcore_axis_name="s", num_cores=NC, num_subcores=NS
    )

    @functools.partial(
        pl.kernel,
        out_type=jax.ShapeDtypeStruct((B * T, EMBED_DIM), jnp.float32),
        mesh=mesh,
        scratch_types=[
            pltpu.VMEM((T, EMBED_DIM), jnp.float32),
            pltpu.VMEM((GPW, T), jnp.float32),
            pltpu.VMEM((GPW, T), jnp.int32),
            pltpu.VMEM((GPW, T), jnp.int32),
            pltpu.SemaphoreType.DMA,
        ],
    )
    def _sc_emit(z_hbm, xc_hbm, table_hbm, emb_hbm,
                 table_v, z_v, xc_v, dest_v, sem):
        wid = lax.axis_index("s") * NC + lax.axis_index("c")
        base = wid * GPW
        pltpu.sync_copy(z_hbm.at[pl.ds(base, GPW)], z_v)
        pltpu.sync_copy(xc_hbm.at[pl.ds(base, GPW)], xc_v)
        pltpu.sync_copy(table_hbm, table_v)

        iota = lax.iota(jnp.int32, L)
        rot_idx = [(iota + r) & (L - 1) for r in range(L)]
        rot_lt = [((iota + r) & (L - 1)) < iota for r in range(L)]

        def graph_body(g, carry):
            ks = []
            for c in range(NV):
                zc = z_v[g, pl.ds(c * L, L)]
                xcc = xc_v[g, pl.ds(c * L, L)]
                ks.append(jnp.where(xcc == 0, zc, jnp.inf))
            ranks = [jnp.zeros((L,), jnp.int32) for _ in range(NV)]
            for r in range(L):
                kdr = [_permute(ks[d], rot_idx[r]) for d in range(NV)]
                for c in range(NV):
                    for d in range(NV):
                        if d < c:
                            cond = kdr[d] <= ks[c]
                        elif d > c:
                            cond = kdr[d] < ks[c]
                        else:
                            if r == 0:
                                continue
                            cond = (kdr[d] < ks[c]) | (
                                (kdr[d] == ks[c]) & rot_lt[r]
                            )
                        ranks[c] = ranks[c] + jnp.where(cond, 1, 0)
            rowbase = (base + g) * T
            for c in range(NV):
                dest_v[g, pl.ds(c * L, L)] = ranks[c] + rowbase
            pltpu.async_copy(table_v, emb_hbm.at[dest_v.at[g]], sem)
            return carry

        lax.fori_loop(0, GPW, graph_body, 0)

        def drain_body(g, carry):
            pltpu.make_async_copy(table_v, emb_hbm.at[dest_v.at[g]], sem).wait()
            return carry

        lax.fori_loop(0, GPW, drain_body, 0)

    return _sc_emit


def kernel(pos_clone, x, table):
    z2d = pos_clone[:, :, 2]
    xc = x[:, :, 0].astype(jnp.int32)
    emb_flat = _get_sc_kernel()(z2d, xc, table)
    sp = _tc_sp(z2d.T, xc.T)
    return (emb_flat.reshape(B, T, EMBED_DIM), sp)
